# baseline (speedup 1.0000x reference)
import functools

import jax
import jax.numpy as jnp
from jax import lax
from jax.experimental import pallas as pl
from jax.experimental.pallas import tpu as pltpu
from jax.experimental.pallas import tpu_sc as plsc

_NC = 2
_NS = 16
_NW = _NC * _NS


def _sc_gather_rows(emb, idx):
    B = idx.shape[0]
    D = emb.shape[1]
    b_per_w = B // _NW
    mesh = plsc.VectorSubcoreMesh(core_axis_name="c", subcore_axis_name="s")

    @functools.partial(
        pl.kernel,
        mesh=mesh,
        compiler_params=pltpu.CompilerParams(use_tc_tiling_on_sc=False),
        out_type=jax.ShapeDtypeStruct((B, D), jnp.float32),
        scratch_types=[
            pltpu.VMEM((b_per_w,), jnp.int32),
            pltpu.VMEM((b_per_w, D), jnp.float32),
            pltpu.SemaphoreType.DMA,
        ],
    )
    def gather_kernel(table_hbm, idx_hbm, out_hbm, idx_v, rows_v, sem):
        wid = lax.axis_index("s") * _NC + lax.axis_index("c")
        base = wid * b_per_w
        pltpu.sync_copy(idx_hbm.at[pl.ds(base, b_per_w)], idx_v)

        @pl.loop(0, b_per_w // 16)
        def _(g):
            vec = idx_v[pl.ds(g * 16, 16)]
            for j in range(16):
                pltpu.make_async_copy(
                    table_hbm.at[pl.ds(vec[j], 1)],
                    rows_v.at[pl.ds(g * 16 + j, 1)],
                    sem,
                ).start()

        @pl.loop(0, b_per_w)
        def _(k):
            pltpu.make_async_copy(
                table_hbm.at[pl.ds(0, 1)], rows_v.at[pl.ds(0, 1)], sem
            ).wait()

        pltpu.sync_copy(rows_v, out_hbm.at[pl.ds(base, b_per_w)])

    return gather_kernel(emb, idx)


def _mlp_kernel(x_ref, w1_ref, b1_ref, w2_ref, b2_ref, w3_ref, b3_ref,
                wo_ref, bo_ref, o_ref):
    h = jnp.dot(x_ref[...], w1_ref[...], preferred_element_type=jnp.float32)
    h = jnp.maximum(h + b1_ref[...], 0.0)
    h = jnp.dot(h, w2_ref[...], preferred_element_type=jnp.float32)
    h = jnp.maximum(h + b2_ref[...], 0.0)
    h = jnp.dot(h, w3_ref[...], preferred_element_type=jnp.float32)
    h = jnp.maximum(h + b3_ref[...], 0.0)
    o_ref[...] = (
        jnp.dot(h, wo_ref[...], preferred_element_type=jnp.float32)
        + bo_ref[...]
    )


def _tc_mlp(embeds, W1T, b1, W2T, b2, W3T, b3, WoT, bo, blk):
    B, D = embeds.shape
    H = W2T.shape[0]
    O = WoT.shape[1]
    full = lambda shape: pl.BlockSpec(shape, lambda i: (0, 0))
    return pl.pallas_call(
        _mlp_kernel,
        grid=(B // blk,),
        in_specs=[
            pl.BlockSpec((blk, D), lambda i: (i, 0)),
            full((D, H)), full((1, H)),
            full((H, H)), full((1, H)),
            full((H, H)), full((1, H)),
            full((H, O)), full((1, O)),
        ],
        out_specs=pl.BlockSpec((blk, O), lambda i: (i, 0)),
        out_shape=jax.ShapeDtypeStruct((B, O), jnp.float32),
    )(embeds, W1T, b1, W2T, b2, W3T, b3, WoT, bo)


def kernel(x, emb, W1, b1, W2, b2, W3, b3, Wm, bm, Wc, bc, Ws, bs):
    idx = x[:, 0].astype(jnp.int32)
    embeds = _sc_gather_rows(emb, idx)
    WoT = jnp.concatenate([Wm, Wc, Ws], axis=0).T
    bo = jnp.concatenate([bm, bc, bs], axis=0)[None, :]
    return _tc_mlp(embeds, W1.T, b1[None, :], W2.T, b2[None, :],
                   W3.T, b3[None, :], WoT, bo, blk=2048)


# speedup vs baseline: 1.0016x; 1.0016x over previous
import functools

import jax
import jax.numpy as jnp
from jax import lax
from jax.experimental import pallas as pl
from jax.experimental.pallas import tpu as pltpu
from jax.experimental.pallas import tpu_sc as plsc

_NC = 2
_NS = 16
_NW = _NC * _NS


_DP = 8


def _sc_gather_rows(flat, idx):
    B = idx.shape[0]
    b_per_w = B // _NW
    mesh = plsc.VectorSubcoreMesh(core_axis_name="c", subcore_axis_name="s")

    @functools.partial(
        pl.kernel,
        mesh=mesh,
        out_type=jax.ShapeDtypeStruct((B * _DP,), jnp.float32),
        scratch_types=[
            pltpu.VMEM((b_per_w,), jnp.int32),
            pltpu.VMEM((b_per_w * _DP,), jnp.float32),
            pltpu.SemaphoreType.DMA,
        ],
    )
    def gather_kernel(flat_hbm, idx_hbm, out_hbm, idx_v, rows_v, sem):
        wid = lax.axis_index("s") * _NC + lax.axis_index("c")
        base = wid * b_per_w
        pltpu.sync_copy(idx_hbm.at[pl.ds(base, b_per_w)], idx_v)

        @pl.loop(0, b_per_w // 16)
        def _(g):
            vec = idx_v[pl.ds(g * 16, 16)]
            for j in range(16):
                k = g * 16 + j
                pltpu.make_async_copy(
                    flat_hbm.at[pl.ds(vec[j] * _DP, _DP)],
                    rows_v.at[pl.ds(k * _DP, _DP)],
                    sem,
                ).start()

        @pl.loop(0, b_per_w)
        def _(k):
            pltpu.make_async_copy(
                flat_hbm.at[pl.ds(0, _DP)], rows_v.at[pl.ds(0, _DP)], sem
            ).wait()

        pltpu.sync_copy(rows_v, out_hbm.at[pl.ds(base * _DP, b_per_w * _DP)])

    return gather_kernel(flat, idx)


def _mlp_kernel(x_ref, w1_ref, b1_ref, w2_ref, b2_ref, w3_ref, b3_ref,
                wo_ref, bo_ref, o_ref):
    h = jnp.dot(x_ref[...], w1_ref[...], preferred_element_type=jnp.float32)
    h = jnp.maximum(h + b1_ref[...], 0.0)
    h = jnp.dot(h, w2_ref[...], preferred_element_type=jnp.float32)
    h = jnp.maximum(h + b2_ref[...], 0.0)
    h = jnp.dot(h, w3_ref[...], preferred_element_type=jnp.float32)
    h = jnp.maximum(h + b3_ref[...], 0.0)
    o_ref[...] = (
        jnp.dot(h, wo_ref[...], preferred_element_type=jnp.float32)
        + bo_ref[...]
    )


def _tc_mlp(embeds, W1T, b1, W2T, b2, W3T, b3, WoT, bo, blk):
    B, D = embeds.shape
    H = W2T.shape[0]
    O = WoT.shape[1]
    full = lambda shape: pl.BlockSpec(shape, lambda i: (0, 0))
    return pl.pallas_call(
        _mlp_kernel,
        grid=(B // blk,),
        in_specs=[
            pl.BlockSpec((blk, D), lambda i: (i, 0)),
            full((D, H)), full((1, H)),
            full((H, H)), full((1, H)),
            full((H, H)), full((1, H)),
            full((H, O)), full((1, O)),
        ],
        out_specs=pl.BlockSpec((blk, O), lambda i: (i, 0)),
        out_shape=jax.ShapeDtypeStruct((B, O), jnp.float32),
    )(embeds, W1T, b1, W2T, b2, W3T, b3, WoT, bo)


def kernel(x, emb, W1, b1, W2, b2, W3, b3, Wm, bm, Wc, bc, Ws, bs):
    B = x.shape[0]
    D = emb.shape[1]
    idx = x[:, 0].astype(jnp.int32)
    flat = jnp.pad(emb, ((0, 0), (0, _DP - D))).reshape(-1)
    embeds = _sc_gather_rows(flat, idx).reshape(B, _DP)
    W1T = jnp.pad(W1.T, ((0, _DP - D), (0, 0)))
    WoT = jnp.concatenate([Wm, Wc, Ws], axis=0).T
    bo = jnp.concatenate([bm, bc, bs], axis=0)[None, :]
    return _tc_mlp(embeds, W1T, b1[None, :], W2.T, b2[None, :],
                   W3.T, b3[None, :], WoT, bo, blk=2048)


# speedup vs baseline: 12.1420x; 12.1230x over previous
import functools

import jax
import jax.numpy as jnp
from jax import lax
from jax.experimental import pallas as pl
from jax.experimental.pallas import tpu as pltpu
from jax.experimental.pallas import tpu_sc as plsc

_NC = 2
_NS = 16
_NW = _NC * _NS


def _sc_gather_cols(embT, idx):
    D, V = embT.shape
    B = idx.shape[0]
    b_per_w = B // _NW
    mesh = plsc.VectorSubcoreMesh(core_axis_name="c", subcore_axis_name="s")

    @functools.partial(
        pl.kernel,
        mesh=mesh,
        compiler_params=pltpu.CompilerParams(needs_layout_passes=False),
        out_type=jax.ShapeDtypeStruct((D, B), jnp.float32),
        scratch_types=[
            pltpu.VMEM((b_per_w,), jnp.int32),
            pltpu.VMEM((D, 64 * 128), jnp.float32),
            pltpu.VMEM((D, b_per_w), jnp.float32),
            pltpu.SemaphoreType.DMA,
        ],
    )
    def gather_kernel(tab_hbm, idx_hbm, out_hbm, idx_v, win_v, cols_v, sem):
        wid = lax.axis_index("s") * _NC + lax.axis_index("c")
        base = wid * b_per_w
        pltpu.sync_copy(idx_hbm.at[pl.ds(base, b_per_w)], idx_v)

        iota16 = lax.iota(jnp.int32, 16)
        R = 64

        @pl.loop(0, b_per_w // R)
        def _(r):
            for g in range(R // 16):
                vec = idx_v[pl.ds(r * R + g * 16, 16)]
                alv = (vec >> 7) << 7
                for j in range(16):
                    k = g * 16 + j
                    al = pl.multiple_of(alv[j], 128)
                    pltpu.make_async_copy(
                        tab_hbm.at[:, pl.ds(al, 128)],
                        win_v.at[:, pl.ds(k * 128, 128)],
                        sem,
                    ).start()

            @pl.loop(0, R)
            def _(k):
                pltpu.make_async_copy(
                    tab_hbm.at[:, pl.ds(0, 128)],
                    win_v.at[:, pl.ds(0, 128)],
                    sem,
                ).wait()

            for g in range(R // 16):
                vec = idx_v[pl.ds(r * R + g * 16, 16)]
                pos = g * (16 * 128) + iota16 * 128 + (vec & 127)
                for c in range(D):
                    row = jnp.full((16,), c, dtype=jnp.int32)
                    vals = plsc.load_gather(win_v, [row, pos])
                    cols_v[c, pl.ds(r * R + g * 16, 16)] = vals

        pltpu.sync_copy(cols_v, out_hbm.at[:, pl.ds(base, b_per_w)])

    return gather_kernel(embT, idx)


def _mlp_kernel(xT_ref, w1_ref, b1_ref, w2_ref, b2_ref, w3_ref, b3_ref,
                wo_ref, bo_ref, o_ref):
    h = jnp.dot(w1_ref[...], xT_ref[...], preferred_element_type=jnp.float32)
    h = jnp.maximum(h + b1_ref[...], 0.0)
    h = jnp.dot(w2_ref[...], h, preferred_element_type=jnp.float32)
    h = jnp.maximum(h + b2_ref[...], 0.0)
    h = jnp.dot(w3_ref[...], h, preferred_element_type=jnp.float32)
    h = jnp.maximum(h + b3_ref[...], 0.0)
    o = jnp.dot(wo_ref[...], h, preferred_element_type=jnp.float32)
    o_ref[...] = (o + bo_ref[...]).T


def _tc_mlp(xT, W1, b1, W2, b2, W3, b3, Wo, bo, blk):
    D, B = xT.shape
    H = W2.shape[0]
    O = Wo.shape[0]
    full = lambda shape: pl.BlockSpec(shape, lambda i: (0, 0))
    return pl.pallas_call(
        _mlp_kernel,
        grid=(B // blk,),
        in_specs=[
            pl.BlockSpec((D, blk), lambda i: (0, i)),
            full((H, D)), full((H, 1)),
            full((H, H)), full((H, 1)),
            full((H, H)), full((H, 1)),
            full((O, H)), full((O, 1)),
        ],
        out_specs=pl.BlockSpec((blk, O), lambda i: (i, 0)),
        out_shape=jax.ShapeDtypeStruct((B, O), jnp.float32),
    )(xT, W1, b1, W2, b2, W3, b3, Wo, bo)


def kernel(x, emb, W1, b1, W2, b2, W3, b3, Wm, bm, Wc, bc, Ws, bs):
    idx = x[:, 0].astype(jnp.int32)
    xT = _sc_gather_cols(emb.T, idx)
    Wo = jnp.concatenate([Wm, Wc, Ws], axis=0)
    bo = jnp.concatenate([bm, bc, bs], axis=0)[:, None]
    return _tc_mlp(xT, W1, b1[:, None], W2, b2[:, None],
                   W3, b3[:, None], Wo, bo, blk=2048)


# speedup vs baseline: 13.9748x; 1.1509x over previous
import functools

import jax
import jax.numpy as jnp
from jax import lax
from jax.experimental import pallas as pl
from jax.experimental.pallas import tpu as pltpu
from jax.experimental.pallas import tpu_sc as plsc

_NC = 2
_NS = 16
_NW = _NC * _NS


def _sc_gather_cols(embT, idx):
    D, V = embT.shape
    B = idx.shape[0]
    b_per_w = B // _NW
    mesh = plsc.VectorSubcoreMesh(core_axis_name="c", subcore_axis_name="s")

    @functools.partial(
        pl.kernel,
        mesh=mesh,
        compiler_params=pltpu.CompilerParams(needs_layout_passes=False),
        out_type=jax.ShapeDtypeStruct((D, B), jnp.float32),
        scratch_types=[
            pltpu.VMEM((b_per_w,), jnp.int32),
            pltpu.VMEM((D, 64 * 128), jnp.float32),
            pltpu.VMEM((D, b_per_w), jnp.float32),
            pltpu.SemaphoreType.DMA,
        ],
    )
    def gather_kernel(tab_hbm, idx_hbm, out_hbm, idx_v, win_v, cols_v, sem):
        wid = lax.axis_index("s") * _NC + lax.axis_index("c")
        base = wid * b_per_w
        pltpu.sync_copy(idx_hbm.at[pl.ds(base, b_per_w)], idx_v)

        iota16 = lax.iota(jnp.int32, 16)
        R = 64

        @pl.loop(0, b_per_w // R)
        def _(r):
            for g in range(R // 16):
                vec = idx_v[pl.ds(r * R + g * 16, 16)]
                alv = (vec >> 7) << 7
                for j in range(16):
                    k = g * 16 + j
                    al = pl.multiple_of(alv[j], 128)
                    pltpu.make_async_copy(
                        tab_hbm.at[:, pl.ds(al, 128)],
                        win_v.at[:, pl.ds(k * 128, 128)],
                        sem,
                    ).start()

            @pl.loop(0, R)
            def _(k):
                pltpu.make_async_copy(
                    tab_hbm.at[:, pl.ds(0, 128)],
                    win_v.at[:, pl.ds(0, 128)],
                    sem,
                ).wait()

            for g in range(R // 16):
                vec = idx_v[pl.ds(r * R + g * 16, 16)]
                pos = g * (16 * 128) + iota16 * 128 + (vec & 127)
                for c in range(D):
                    row = jnp.full((16,), c, dtype=jnp.int32)
                    vals = plsc.load_gather(win_v, [row, pos])
                    cols_v[c, pl.ds(r * R + g * 16, 16)] = vals

        pltpu.sync_copy(cols_v, out_hbm.at[:, pl.ds(base, b_per_w)])

    return gather_kernel(embT, idx)


def _mlp_kernel(xT_ref, w1_ref, b1_ref, w2_ref, b2_ref, w3_ref, b3_ref,
                wo_ref, bo_ref, o_ref):
    h = jnp.dot(w1_ref[...], xT_ref[...], preferred_element_type=jnp.float32)
    h = jnp.maximum(h + b1_ref[...], 0.0)
    h = jnp.dot(w2_ref[...], h, preferred_element_type=jnp.float32)
    h = jnp.maximum(h + b2_ref[...], 0.0)
    h = jnp.dot(w3_ref[...], h, preferred_element_type=jnp.float32)
    h = jnp.maximum(h + b3_ref[...], 0.0)
    o = jnp.dot(wo_ref[...], h, preferred_element_type=jnp.float32)
    o_ref[...] = o + bo_ref[...]


def _tc_mlp(xT, W1, b1, W2, b2, W3, b3, Wo, bo, blk):
    D, B = xT.shape
    H = W2.shape[0]
    O = Wo.shape[0]
    full = lambda shape: pl.BlockSpec(shape, lambda i: (0, 0))
    return pl.pallas_call(
        _mlp_kernel,
        grid=(B // blk,),
        in_specs=[
            pl.BlockSpec((D, blk), lambda i: (0, i)),
            full((H, D)), full((H, 1)),
            full((H, H)), full((H, 1)),
            full((H, H)), full((H, 1)),
            full((O, H)), full((O, 1)),
        ],
        out_specs=pl.BlockSpec((O, blk), lambda i: (0, i)),
        out_shape=jax.ShapeDtypeStruct((O, B), jnp.float32),
    )(xT, W1, b1, W2, b2, W3, b3, Wo, bo)


def kernel(x, emb, W1, b1, W2, b2, W3, b3, Wm, bm, Wc, bc, Ws, bs):
    idx = x[:, 0].astype(jnp.int32)
    xT = _sc_gather_cols(emb.T, idx)
    Wo = jnp.concatenate([Wm, Wc, Ws], axis=0)
    bo = jnp.concatenate([bm, bc, bs], axis=0)[:, None]
    oT = _tc_mlp(xT, W1, b1[:, None], W2, b2[:, None],
                 W3, b3[:, None], Wo, bo, blk=2048)
    return oT.T


# speedup vs baseline: 14.0770x; 1.0073x over previous
import functools

import jax
import jax.numpy as jnp
from jax import lax
from jax.experimental import pallas as pl
from jax.experimental.pallas import tpu as pltpu
from jax.experimental.pallas import tpu_sc as plsc

_NC = 2
_NS = 16
_NW = _NC * _NS


def _sc_gather_cols(embT, idx):
    D, V = embT.shape
    B = idx.shape[0]
    b_per_w = B // _NW
    mesh = plsc.VectorSubcoreMesh(core_axis_name="c", subcore_axis_name="s")

    @functools.partial(
        pl.kernel,
        mesh=mesh,
        compiler_params=pltpu.CompilerParams(needs_layout_passes=False),
        out_type=jax.ShapeDtypeStruct((D, B), jnp.float32),
        scratch_types=[
            pltpu.VMEM((b_per_w,), jnp.int32),
            pltpu.VMEM((D, 32 * 128), jnp.float32),
            pltpu.VMEM((D, 32 * 128), jnp.float32),
            pltpu.VMEM((D, b_per_w), jnp.float32),
            pltpu.SemaphoreType.DMA,
            pltpu.SemaphoreType.DMA,
        ],
    )
    def gather_kernel(tab_hbm, idx_hbm, out_hbm, idx_v, win0, win1, cols_v,
                      sem0, sem1):
        wid = lax.axis_index("s") * _NC + lax.axis_index("c")
        base = wid * b_per_w
        pltpu.sync_copy(idx_hbm.at[pl.ds(base, b_per_w)], idx_v)

        iota16 = lax.iota(jnp.int32, 16)
        R = 32
        n_rounds = b_per_w // R

        def fire(r, win, sem):
            for g in range(R // 16):
                vec = idx_v[pl.ds(r * R + g * 16, 16)]
                alv = (vec >> 7) << 7
                for j in range(16):
                    k = g * 16 + j
                    al = pl.multiple_of(alv[j], 128)
                    pltpu.make_async_copy(
                        tab_hbm.at[:, pl.ds(al, 128)],
                        win.at[:, pl.ds(k * 128, 128)],
                        sem,
                    ).start()

        def drain_extract(r, win, sem):
            @pl.loop(0, R)
            def _(k):
                pltpu.make_async_copy(
                    tab_hbm.at[:, pl.ds(0, 128)],
                    win.at[:, pl.ds(0, 128)],
                    sem,
                ).wait()

            for g in range(R // 16):
                vec = idx_v[pl.ds(r * R + g * 16, 16)]
                pos = g * (16 * 128) + iota16 * 128 + (vec & 127)
                for c in range(D):
                    row = jnp.full((16,), c, dtype=jnp.int32)
                    vals = plsc.load_gather(win, [row, pos])
                    cols_v[c, pl.ds(r * R + g * 16, 16)] = vals

        fire(0, win0, sem0)

        @pl.loop(0, n_rounds // 2)
        def _(p):
            fire(2 * p + 1, win1, sem1)
            drain_extract(2 * p, win0, sem0)

            @pl.when(p < n_rounds // 2 - 1)
            def _():
                fire(2 * p + 2, win0, sem0)

            drain_extract(2 * p + 1, win1, sem1)

        pltpu.sync_copy(cols_v, out_hbm.at[:, pl.ds(base, b_per_w)])

    return gather_kernel(embT, idx)


def _mlp_kernel(xT_ref, w1_ref, b1_ref, w2_ref, b2_ref, w3_ref, b3_ref,
                wo_ref, bo_ref, o_ref):
    h = jnp.dot(w1_ref[...], xT_ref[...], preferred_element_type=jnp.float32)
    h = jnp.maximum(h + b1_ref[...], 0.0)
    h = jnp.dot(w2_ref[...], h, preferred_element_type=jnp.float32)
    h = jnp.maximum(h + b2_ref[...], 0.0)
    h = jnp.dot(w3_ref[...], h, preferred_element_type=jnp.float32)
    h = jnp.maximum(h + b3_ref[...], 0.0)
    o = jnp.dot(wo_ref[...], h, preferred_element_type=jnp.float32)
    o_ref[...] = o + bo_ref[...]


def _tc_mlp(xT, W1, b1, W2, b2, W3, b3, Wo, bo, blk):
    D, B = xT.shape
    H = W2.shape[0]
    O = Wo.shape[0]
    full = lambda shape: pl.BlockSpec(shape, lambda i: (0, 0))
    return pl.pallas_call(
        _mlp_kernel,
        grid=(B // blk,),
        in_specs=[
            pl.BlockSpec((D, blk), lambda i: (0, i)),
            full((H, D)), full((H, 1)),
            full((H, H)), full((H, 1)),
            full((H, H)), full((H, 1)),
            full((O, H)), full((O, 1)),
        ],
        out_specs=pl.BlockSpec((O, blk), lambda i: (0, i)),
        out_shape=jax.ShapeDtypeStruct((O, B), jnp.float32),
    )(xT, W1, b1, W2, b2, W3, b3, Wo, bo)


def kernel(x, emb, W1, b1, W2, b2, W3, b3, Wm, bm, Wc, bc, Ws, bs):
    idx = x[:, 0].astype(jnp.int32)
    xT = _sc_gather_cols(emb.T, idx)
    Wo = jnp.concatenate([Wm, Wc, Ws], axis=0)
    bo = jnp.concatenate([bm, bc, bs], axis=0)[:, None]
    oT = _tc_mlp(xT, W1, b1[:, None], W2, b2[:, None],
                 W3, b3[:, None], Wo, bo, blk=2048)
    return oT.T
